# baseline (speedup 1.0000x reference)
import functools

import jax
import jax.numpy as jnp
from jax.experimental import pallas as pl
from jax.experimental.pallas import tpu as pltpu
from jax.experimental.pallas import tpu_sc as plsc

B = 4
S = 8192
D = 768
N = B * S
C = 128
NC = D // C
W = 128
NWIN = N // W

TC_ROWS = 2048


def _gather_sc(fidx, table_flat):
    mesh = plsc.VectorSubcoreMesh(core_axis_name="c", subcore_axis_name="s")

    @functools.partial(
        pl.kernel,
        out_type=jax.ShapeDtypeStruct((N, D), jnp.float32),
        mesh=mesh,
    )
    def k(i_hbm, t_hbm, o_hbm):
        def body(i_vmem, o_vmem):
            pltpu.sync_copy(t_hbm.at[i_vmem.at[0]], o_vmem)

        pltpu.emit_pipeline(
            body,
            grid=(NWIN, NC),
            in_specs=[pl.BlockSpec((1, W), lambda i, j: (i * NC + j, 0))],
            out_specs=[pl.BlockSpec((W, C), lambda i, j: (i, j))],
            core_axis_name=("c", "s"),
            dimension_semantics=(pltpu.PARALLEL, pltpu.PARALLEL),
        )(i_hbm, o_hbm)

    return k(fidx, table_flat)


def _add_tc(x2d, pe2d):
    def body(x_ref, pe_ref, o_ref):
        o_ref[...] = x_ref[...] + pe_ref[...]

    return pl.pallas_call(
        body,
        out_shape=jax.ShapeDtypeStruct((N, D), jnp.float32),
        grid=(N // TC_ROWS,),
        in_specs=[
            pl.BlockSpec((TC_ROWS, D), lambda i: (i, 0)),
            pl.BlockSpec((TC_ROWS, D), lambda i: (i, 0)),
        ],
        out_specs=pl.BlockSpec((TC_ROWS, D), lambda i: (i, 0)),
    )(x2d, pe2d)


def kernel(x, position, pos_table):
    x2d = x.reshape(N, D)
    pos = position.reshape(NWIN, W).astype(jnp.int32)
    fidx = (pos[:, None, :] * NC + jnp.arange(NC, dtype=jnp.int32)[None, :, None])
    fidx = fidx.reshape(NWIN * NC, W)
    table_flat = pos_table.reshape(8192 * NC, C)
    pe2d = _gather_sc(fidx, table_flat)
    out = _add_tc(x2d, pe2d)
    return out.reshape(B, S, D)


# speedup vs baseline: 1.0344x; 1.0344x over previous
import functools

import jax
import jax.numpy as jnp
from jax.experimental import pallas as pl
from jax.experimental.pallas import tpu as pltpu
from jax.experimental.pallas import tpu_sc as plsc

B = 4
S = 8192
D = 768
N = B * S
C = 128
NC = D // C
W = 128
NWIN = N // W

K = 2
NK = N // K
NWK = NWIN // K

TC_ROWS = 2048
TB = NK // TC_ROWS


def _gather_sc(fidx, table_flat):
    mesh = plsc.VectorSubcoreMesh(core_axis_name="c", subcore_axis_name="s")

    @functools.partial(
        pl.kernel,
        out_type=jax.ShapeDtypeStruct((NK, D), jnp.float32),
        mesh=mesh,
    )
    def k(i_hbm, t_hbm, o_hbm):
        def body(i_vmem, o_vmem):
            pltpu.sync_copy(t_hbm.at[i_vmem.at[0]], o_vmem)

        pltpu.emit_pipeline(
            body,
            grid=(NWK, NC),
            in_specs=[pl.BlockSpec((1, W), lambda i, j: (i * NC + j, 0))],
            out_specs=[pl.BlockSpec((W, C), lambda i, j: (i, j))],
            core_axis_name=("c", "s"),
            dimension_semantics=(pltpu.PARALLEL, pltpu.PARALLEL),
        )(i_hbm, o_hbm)

    return k(fidx, table_flat)


def _add_first(x2d, pe0):
    def body(x_ref, pe_ref, o_ref):
        o_ref[...] = x_ref[...] + pe_ref[...]

    return pl.pallas_call(
        body,
        out_shape=jax.ShapeDtypeStruct((N, D), jnp.float32),
        grid=(TB,),
        in_specs=[
            pl.BlockSpec((TC_ROWS, D), lambda i: (i, 0)),
            pl.BlockSpec((TC_ROWS, D), lambda i: (i, 0)),
        ],
        out_specs=pl.BlockSpec((TC_ROWS, D), lambda i: (i, 0)),
    )(x2d, pe0)


def _add_second(prev, x2d, pe1):
    def body(prev_ref, x_ref, pe_ref, o_ref):
        o_ref[...] = x_ref[...] + pe_ref[...]

    return pl.pallas_call(
        body,
        out_shape=jax.ShapeDtypeStruct((N, D), jnp.float32),
        grid=(TB,),
        in_specs=[
            pl.BlockSpec(memory_space=pltpu.MemorySpace.HBM),
            pl.BlockSpec((TC_ROWS, D), lambda i: (i + TB, 0)),
            pl.BlockSpec((TC_ROWS, D), lambda i: (i, 0)),
        ],
        out_specs=pl.BlockSpec((TC_ROWS, D), lambda i: (i + TB, 0)),
        input_output_aliases={0: 0},
    )(prev, x2d, pe1)


def kernel(x, position, pos_table):
    x2d = x.reshape(N, D)
    pos = position.reshape(NWIN, W).astype(jnp.int32)
    fidx = (pos[:, None, :] * NC + jnp.arange(NC, dtype=jnp.int32)[None, :, None])
    fidx = fidx.reshape(K, NWK * NC, W)
    table_flat = pos_table.reshape(8192 * NC, C)
    pe0 = _gather_sc(fidx[0], table_flat)
    pe1 = _gather_sc(fidx[1], table_flat)
    out = _add_first(x2d, pe0)
    out = _add_second(out, x2d, pe1)
    return out.reshape(B, S, D)
